# baseline (speedup 1.0000x reference)
import functools

import jax
import jax.numpy as jnp
from jax import lax
from jax.experimental import pallas as pl
from jax.experimental.pallas import tpu as pltpu
from jax.experimental.pallas import tpu_sc as plsc

DIM = 64
VOCAB_P2 = 1002
BATCH = 16384
SEQ = 50
SEQ1 = SEQ + 1
NC, NS = 2, 16
NW = NC * NS
SEQ_PER_W = BATCH // NW
NSQ = 16
NGRP = SEQ_PER_W // NSQ


def _gather_body(table_hbm, idx_hbm, out_hbm, idx_v, rows_v, isem, gsem, ssem):
    wid = lax.axis_index("s") * NC + lax.axis_index("c")
    base_seq = wid * SEQ_PER_W

    pltpu.sync_copy(idx_hbm.at[pl.ds(base_seq, NSQ)], idx_v.at[0])

    def body(i, carry):
        b = lax.rem(i, 2)
        pb = 1 - b
        seq0 = base_seq + i * NSQ

        @pl.when(i >= 1)
        def _wait_idx():
            pltpu.make_async_copy(
                idx_hbm.at[pl.ds(base_seq, NSQ)], idx_v.at[b], isem
            ).wait()

        gathers = [
            pltpu.async_copy(
                table_hbm.at[idx_v.at[b].at[j]], rows_v.at[b].at[j], gsem
            )
            for j in range(NSQ)
        ]

        @pl.when(i < NGRP - 1)
        def _prefetch_idx():
            pltpu.async_copy(
                idx_hbm.at[pl.ds(seq0 + NSQ, NSQ)], idx_v.at[pb], isem
            )

        @pl.when(i >= 1)
        def _wait_store():
            pltpu.make_async_copy(
                rows_v.at[pb], out_hbm.at[pl.ds(base_seq, NSQ)], ssem
            ).wait()

        for c in gathers:
            c.wait()
        pltpu.async_copy(rows_v.at[b], out_hbm.at[pl.ds(seq0, NSQ)], ssem)
        return carry

    lax.fori_loop(0, NGRP, body, 0)
    lb = (NGRP - 1) % 2
    pltpu.make_async_copy(
        rows_v.at[lb], out_hbm.at[pl.ds(base_seq, NSQ)], ssem
    ).wait()


_sc_gather = functools.partial(
    pl.kernel,
    out_type=jax.ShapeDtypeStruct((BATCH, SEQ1, DIM), jnp.float32),
    mesh=plsc.VectorSubcoreMesh(core_axis_name="c", subcore_axis_name="s"),
    scratch_types=[
        pltpu.VMEM((2, NSQ, SEQ1), jnp.int32),
        pltpu.VMEM((2, NSQ, SEQ1, DIM), jnp.float32),
        pltpu.SemaphoreType.DMA,
        pltpu.SemaphoreType.DMA,
        pltpu.SemaphoreType.DMA,
    ],
    compiler_params=pltpu.CompilerParams(use_tc_tiling_on_sc=False),
)(_gather_body)


def kernel(tokens, charges, aa_table, charge_table):
    aa_table = aa_table.at[0].set(0.0)
    table = jnp.concatenate([aa_table, charge_table], axis=0)
    cidx = jnp.concatenate(
        [charges.astype(jnp.int32)[:, None] + VOCAB_P2, tokens.astype(jnp.int32)],
        axis=1,
    )
    return _sc_gather(table, cidx)


# speedup vs baseline: 1.3795x; 1.3795x over previous
import functools

import jax
import jax.numpy as jnp
from jax import lax
from jax.experimental import pallas as pl
from jax.experimental.pallas import tpu as pltpu
from jax.experimental.pallas import tpu_sc as plsc

DIM = 64
NTAB = 1012
VOCAB_P2 = 1002
BATCH = 16384
SEQ = 50
SEQ1 = SEQ + 1
NC, NS = 2, 16
NW = NC * NS
SEQ_PER_W = BATCH // NW
NSQ = 8
NGRP = SEQ_PER_W // NSQ


def _gather_body(tab_hbm, idx_hbm, out_hbm,
                 tab_sp, idx_v, rows_v, isem, gsem, ssem):
    sid = lax.axis_index("s")
    wid = sid * NC + lax.axis_index("c")
    base_seq = wid * SEQ_PER_W

    @pl.when(sid == 0)
    def _fill_table():
        pltpu.sync_copy(tab_hbm, tab_sp)

    plsc.subcore_barrier()

    pltpu.sync_copy(idx_hbm.at[pl.ds(base_seq, NSQ)], idx_v.at[0])

    def body(i, carry):
        b = lax.rem(i, 2)
        pb = 1 - b
        seq0 = base_seq + i * NSQ

        @pl.when(i >= 1)
        def _wait_idx():
            pltpu.make_async_copy(
                idx_hbm.at[pl.ds(base_seq, NSQ)], idx_v.at[b], isem
            ).wait()

        gathers = [
            pltpu.async_copy(
                tab_sp.at[idx_v.at[b].at[j]], rows_v.at[b].at[j], gsem
            )
            for j in range(NSQ)
        ]

        @pl.when(i < NGRP - 1)
        def _prefetch_idx():
            pltpu.async_copy(
                idx_hbm.at[pl.ds(seq0 + NSQ, NSQ)], idx_v.at[pb], isem
            )

        @pl.when(i >= 1)
        def _wait_store():
            pltpu.make_async_copy(
                rows_v.at[pb], out_hbm.at[pl.ds(base_seq, NSQ)], ssem
            ).wait()

        for c in gathers:
            c.wait()
        pltpu.async_copy(rows_v.at[b], out_hbm.at[pl.ds(seq0, NSQ)], ssem)
        return carry

    lax.fori_loop(0, NGRP, body, 0)
    lb = (NGRP - 1) % 2
    pltpu.make_async_copy(
        rows_v.at[lb], out_hbm.at[pl.ds(base_seq, NSQ)], ssem
    ).wait()


_sc_gather = functools.partial(
    pl.kernel,
    out_type=jax.ShapeDtypeStruct((BATCH, SEQ1, DIM), jnp.float32),
    mesh=plsc.VectorSubcoreMesh(core_axis_name="c", subcore_axis_name="s"),
    scratch_types=[
        pltpu.VMEM_SHARED((NTAB, DIM), jnp.float32),
        pltpu.VMEM((2, NSQ, SEQ1), jnp.int32),
        pltpu.VMEM((2, NSQ, SEQ1, DIM), jnp.float32),
        pltpu.SemaphoreType.DMA,
        pltpu.SemaphoreType.DMA,
        pltpu.SemaphoreType.DMA,
    ],
    compiler_params=pltpu.CompilerParams(use_tc_tiling_on_sc=False),
)(_gather_body)


def kernel(tokens, charges, aa_table, charge_table):
    aa_table = aa_table.at[0].set(0.0)
    table = jnp.concatenate([aa_table, charge_table], axis=0)
    cidx = jnp.concatenate(
        [charges.astype(jnp.int32)[:, None] + VOCAB_P2, tokens.astype(jnp.int32)],
        axis=1,
    )
    return _sc_gather(table, cidx)


# speedup vs baseline: 1.7360x; 1.2584x over previous
import functools

import jax
import jax.numpy as jnp
from jax import lax
from jax.experimental import pallas as pl
from jax.experimental.pallas import tpu as pltpu
from jax.experimental.pallas import tpu_sc as plsc

DIM = 64
NTAB = 1012
VOCAB_P2 = 1002
BATCH = 16384
SEQ = 50
SEQ1 = SEQ + 1
NC, NS = 2, 16
NW = NC * NS
SEQ_PER_W = BATCH // NW
NSQ = 8
SEQP = 56
NGRP = SEQ_PER_W // NSQ


def _gather_body(tab_hbm, idx_hbm, out_hbm,
                 tab_sp, idx_v, rows_v, isem, gsem, ssem):
    sid = lax.axis_index("s")
    wid = sid * NC + lax.axis_index("c")
    base_seq = wid * SEQ_PER_W

    @pl.when(sid == 0)
    def _fill_table():
        pltpu.sync_copy(tab_hbm, tab_sp)

    plsc.subcore_barrier()

    base_i = wid * (SEQ_PER_W * SEQP)
    gi = NSQ * SEQP

    pltpu.sync_copy(idx_hbm.at[pl.ds(base_i, gi)], idx_v.at[pl.ds(0, gi)])

    def body(i, carry):
        b = lax.rem(i, 2)
        pb = 1 - b
        seq0 = base_seq + i * NSQ

        boff = b * gi

        @pl.when(i >= 1)
        def _wait_idx():
            pltpu.make_async_copy(
                idx_hbm.at[pl.ds(base_i, gi)], idx_v.at[pl.ds(boff, gi)], isem
            ).wait()

        gathers = [
            pltpu.async_copy(
                tab_sp.at[idx_v.at[pl.ds(boff + SEQP * j, SEQ1)]],
                rows_v.at[b].at[j], gsem,
            )
            for j in range(NSQ)
        ]

        @pl.when(i < NGRP - 1)
        def _prefetch_idx():
            pltpu.async_copy(
                idx_hbm.at[pl.ds(base_i + (i + 1) * gi, gi)],
                idx_v.at[pl.ds((1 - b) * gi, gi)], isem,
            )

        @pl.when(i >= 1)
        def _wait_store():
            pltpu.make_async_copy(
                rows_v.at[pb], out_hbm.at[pl.ds(base_seq, NSQ)], ssem
            ).wait()

        for c in gathers:
            c.wait()
        pltpu.async_copy(rows_v.at[b], out_hbm.at[pl.ds(seq0, NSQ)], ssem)
        return carry

    lax.fori_loop(0, NGRP, body, 0)
    lb = (NGRP - 1) % 2
    pltpu.make_async_copy(
        rows_v.at[lb], out_hbm.at[pl.ds(base_seq, NSQ)], ssem
    ).wait()


_sc_gather = functools.partial(
    pl.kernel,
    out_type=jax.ShapeDtypeStruct((BATCH, SEQ1, DIM), jnp.float32),
    mesh=plsc.VectorSubcoreMesh(core_axis_name="c", subcore_axis_name="s"),
    scratch_types=[
        pltpu.VMEM_SHARED((NTAB, DIM), jnp.float32),
        pltpu.VMEM((2 * NSQ * SEQP,), jnp.int32),
        pltpu.VMEM((2, NSQ, SEQ1, DIM), jnp.float32),
        pltpu.SemaphoreType.DMA,
        pltpu.SemaphoreType.DMA,
        pltpu.SemaphoreType.DMA,
    ],
    compiler_params=pltpu.CompilerParams(use_tc_tiling_on_sc=True),
)(_gather_body)


def kernel(tokens, charges, aa_table, charge_table):
    aa_table = aa_table.at[0].set(0.0)
    table = jnp.concatenate([aa_table, charge_table], axis=0)
    cidx = jnp.concatenate(
        [charges.astype(jnp.int32)[:, None] + VOCAB_P2, tokens.astype(jnp.int32)],
        axis=1,
    )
    cidx = jnp.pad(cidx, ((0, 0), (0, SEQP - SEQ1))).reshape(-1)
    return _sc_gather(table, cidx)
